# baseline (speedup 1.0000x reference)
import functools

import jax
import jax.numpy as jnp
from jax import lax
from jax.experimental import pallas as pl
from jax.experimental.pallas import tpu as pltpu
from jax.experimental.pallas import tpu_sc as plsc

_TEMP = 0.1
_MOM = 0.1
_V = 100000
_D = 64
_B = 1024
_BR = 1000
_NB = _V // _BR


_G_NW = 32
_G_PER = _B // _G_NW


def _gather_body(lab_ref, feat_ref, g_ref, idx_v, rows_v, sem):
    c = lax.axis_index("c")
    s = lax.axis_index("s")
    wid = s * 2 + c
    base = wid * _G_PER
    pltpu.sync_copy(lab_ref.at[pl.ds(base, _G_PER)], idx_v)
    pltpu.async_copy(feat_ref.at[idx_v], rows_v, sem).wait()
    pltpu.sync_copy(rows_v, g_ref.at[pl.ds(base, _G_PER)])


@functools.cache
def _sc_gather():
    return pl.kernel(
        _gather_body,
        out_type=jax.ShapeDtypeStruct((_B, _D), jnp.float32),
        mesh=plsc.VectorSubcoreMesh(core_axis_name="c", subcore_axis_name="s"),
        scratch_types=[
            pltpu.VMEM((_G_PER,), jnp.int32),
            pltpu.VMEM((_G_PER, _D), jnp.float32),
            pltpu.SemaphoreType.DMA,
        ],
        compiler_params=pltpu.CompilerParams(use_tc_tiling_on_sc=False),
    )


def _main_body(xT_ref, gT_ref, lc_ref, lr_ref, f_ref, p_ref, updT_ref,
               xsT_ref, down_ref):
    i = pl.program_id(0)

    @pl.when(i == 0)
    def _init():
        xT = xT_ref[...]
        n = jnp.sqrt(jnp.sum(xT * xT, axis=0, keepdims=True))
        xnT = xT / (n + 1e-12)
        xsT_ref[...] = xnT / _TEMP
        down_ref[...] = jnp.zeros_like(down_ref)
        gT = gT_ref[...]
        u = _MOM * gT + (1.0 - _MOM) * xnT
        un = jnp.sqrt(jnp.sum(u * u, axis=0, keepdims=True))
        u = u / (un + 1e-12)
        lc = lc_ref[...]
        lr = lr_ref[0:1, :]
        eq = lc == lr
        ii = lax.broadcasted_iota(jnp.int32, (_B, _B), 0)
        lo = jnp.max(jnp.where(eq, ii, -1), axis=0, keepdims=True)
        q = (ii == lo).astype(jnp.float32)
        updT_ref[...] = jnp.dot(u, q, precision=lax.Precision.HIGHEST)

    s2 = jnp.dot(f_ref[...], xsT_ref[...])
    e = jnp.exp(s2)
    down_ref[...] += jnp.sum(e, axis=0, keepdims=True)

    @pl.when(i == _NB - 1)
    def _fin():
        dots = jnp.sum(xsT_ref[...] * gT_ref[...], axis=0, keepdims=True)
        p_ref[...] = jnp.exp(dots) / down_ref[...]


_main_call = pl.pallas_call(
    _main_body,
    grid=(_NB,),
    in_specs=[
        pl.BlockSpec((_D, _B), lambda i: (0, 0)),
        pl.BlockSpec((_D, _B), lambda i: (0, 0)),
        pl.BlockSpec((_B, 1), lambda i: (0, 0)),
        pl.BlockSpec((8, _B), lambda i: (0, 0)),
        pl.BlockSpec((_BR, _D), lambda i: (i, 0)),
    ],
    out_specs=[
        pl.BlockSpec((1, _B), lambda i: (0, 0)),
        pl.BlockSpec((_D, _B), lambda i: (0, 0)),
    ],
    out_shape=[
        jax.ShapeDtypeStruct((1, _B), jnp.float32),
        jax.ShapeDtypeStruct((_D, _B), jnp.float32),
    ],
    scratch_shapes=[
        pltpu.VMEM((_D, _B), jnp.float32),
        pltpu.VMEM((1, _B), jnp.float32),
    ],
    compiler_params=pltpu.CompilerParams(
        dimension_semantics=("arbitrary",),
    ),
)


_S_NW = 16
_S_ROWS = _V // _S_NW
_S_CHUNK = 1250
_S_NCH = _S_ROWS // _S_CHUNK
_S_PER = _B // _S_NW


def _scatter_body(feat_ref, upd_ref, lab_ref, out_ref, buf, idx_v, rows_v,
                  sem):
    s = lax.axis_index("s")
    row0 = s * _S_ROWS
    for c in range(_S_NCH):
        b = row0 + c * _S_CHUNK
        pltpu.sync_copy(feat_ref.at[pl.ds(b, _S_CHUNK)], buf)
        pltpu.sync_copy(buf, out_ref.at[pl.ds(b, _S_CHUNK)])
    plsc.subcore_barrier()
    i0 = s * _S_PER
    pltpu.sync_copy(lab_ref.at[pl.ds(i0, _S_PER)], idx_v)
    pltpu.sync_copy(upd_ref.at[pl.ds(i0, _S_PER)], rows_v)
    pltpu.async_copy(rows_v, out_ref.at[idx_v], sem).wait()


@functools.cache
def _sc_scatter():
    return pl.kernel(
        _scatter_body,
        out_type=jax.ShapeDtypeStruct((_V, _D), jnp.float32),
        mesh=plsc.VectorSubcoreMesh(
            core_axis_name="c", subcore_axis_name="s", num_cores=1
        ),
        scratch_types=[
            pltpu.VMEM((_S_CHUNK, _D), jnp.float32),
            pltpu.VMEM((_S_PER,), jnp.int32),
            pltpu.VMEM((_S_PER, _D), jnp.float32),
            pltpu.SemaphoreType.DMA,
        ],
        compiler_params=pltpu.CompilerParams(use_tc_tiling_on_sc=False),
    )


def kernel(inputs, labels, features):
    labels = labels.astype(jnp.int32)
    g = _sc_gather()(labels, features)
    p2, updT = _main_call(
        inputs.T,
        g.T,
        labels[:, None],
        jnp.broadcast_to(labels[None, :], (8, _B)),
        features,
    )
    new_mem = _sc_scatter()(features, updT.T, labels)
    return p2.reshape(_B), new_mem
